# Initial kernel scaffold; baseline (speedup 1.0000x reference)
import jax
import jax.numpy as jnp
from jax.experimental import pallas as pl


def kernel(x, edge_index, neg_edge_index, adapt_W, adapt_b, kW, kb, qW, qb, vW, vb, aW, ab, rel_pri, rel_att, rel_msg, skip, ln_g, ln_b):
    raise NotImplementedError("write your pallas kernel here")


# speedup vs baseline: 18.8407x; 18.8407x over previous
import functools
import math

import jax
import jax.numpy as jnp
import numpy as np
from jax import lax
from jax.experimental import pallas as pl
from jax.experimental.pallas import tpu as pltpu
from jax.experimental.pallas import tpu_sc as plsc

N = 10000
E = 320000
D = 128
H = 8
DK = 16
L = 2

NC = 2
NS = 16
NW = NC * NS

NPAD = 10240
EPW = E // NW
EB = 40
NBLK = EPW // EB
ROWS_PT = NPAD // NS
ZR = 128

E2 = 2 * E
EPW2 = E2 // NW
EB2 = 80
NBLK2 = EPW2 // EB2

RB = 1000
NRB = N // RB

_f32 = jnp.float32


def _perm():
    p = np.zeros(D, np.int32)
    for c in range(D):
        j, l = divmod(c, 16)
        head = l if l < 8 else 15 - l
        dk = j if l < 8 else j + 8
        p[c] = head * 16 + dk
    return p


_P = _perm()


def _tc_adapt_body(x_ref, wa_ref, ba_ref, wq_ref, bq_ref, wk_ref, bk_ref,
                   wv_ref, bv_ref, h_ref, q_ref, k_ref, v_ref):
    xb = x_ref[...]
    pre = xb @ wa_ref[...] + ba_ref[...]
    hb = 0.5 * pre * (1.0 + lax.erf(pre * (1.0 / math.sqrt(2.0))))
    h_ref[...] = hb
    q_ref[...] = hb @ wq_ref[...] + bq_ref[...]
    k_ref[...] = hb @ wk_ref[...] + bk_ref[...]
    v_ref[...] = hb @ wv_ref[...] + bv_ref[...]


def _tc_adapt(x, wa, ba, wq, bq, wk, bk, wv, bv):
    row = pl.BlockSpec((RB, D), lambda i: (i, 0))
    wsp = pl.BlockSpec((D, D), lambda i: (0, 0))
    bsp = pl.BlockSpec((1, D), lambda i: (0, 0))
    out = jax.ShapeDtypeStruct((N, D), _f32)
    return pl.pallas_call(
        _tc_adapt_body,
        grid=(NRB,),
        in_specs=[row, wsp, bsp, wsp, bsp, wsp, bsp, wsp, bsp],
        out_specs=[row, row, row, row],
        out_shape=[out, out, out, out],
    )(x, wa, ba, wq, bq, wk, bk, wv, bv)


def _tc_combine_body(proj, tu_ref, den_ref, h_ref, wa_ref, ab_ref, g_ref,
                     b_ref, sc_ref, *rest):
    tu = tu_ref[0] + tu_ref[1]
    den = den_ref[0] + den_ref[1]
    den = jnp.maximum(jnp.concatenate([den] * (D // 16), axis=-1), 1e-9)
    t = tu / den
    alpha = sc_ref[0, 0]
    out = (t @ wa_ref[...] + ab_ref[...]) * alpha + h_ref[...] * (1.0 - alpha)
    mu = jnp.mean(out, axis=-1, keepdims=True)
    var = jnp.mean((out - mu) * (out - mu), axis=-1, keepdims=True)
    hb = (out - mu) * jax.lax.rsqrt(var + 1e-5) * g_ref[...] + b_ref[...]
    if proj:
        (wq_ref, bq_ref, wk_ref, bk_ref, wv_ref, bv_ref,
         h_out, q_out, k_out, v_out) = rest
        h_out[...] = hb
        q_out[...] = hb @ wq_ref[...] + bq_ref[...]
        k_out[...] = hb @ wk_ref[...] + bk_ref[...]
        v_out[...] = hb @ wv_ref[...] + bv_ref[...]
    else:
        (h_out,) = rest
        h_out[...] = hb


def _tc_combine(tu_parts, den_parts, h, wa, ab, g, b, alpha, proj_w=None):
    row = pl.BlockSpec((RB, D), lambda i: (i, 0))
    wsp = pl.BlockSpec((D, D), lambda i: (0, 0))
    bsp = pl.BlockSpec((1, D), lambda i: (0, 0))
    ssp = pl.BlockSpec((1, 1), lambda i: (0, 0))
    tusp = pl.BlockSpec((2, RB, D), lambda i: (0, i, 0))
    densp = pl.BlockSpec((2, RB, 16), lambda i: (0, i, 0))
    out = jax.ShapeDtypeStruct((N, D), _f32)
    proj = proj_w is not None
    in_specs = [tusp, densp, row, wsp, bsp, bsp, bsp, ssp]
    args = [tu_parts, den_parts, h, wa, ab, g, b, alpha]
    if proj:
        in_specs += [wsp, bsp, wsp, bsp, wsp, bsp]
        args += list(proj_w)
        out_specs, out_shape = [row] * 4, [out] * 4
    else:
        out_specs, out_shape = [row], [out]
    res = pl.pallas_call(
        functools.partial(_tc_combine_body, proj),
        grid=(NRB,),
        in_specs=in_specs,
        out_specs=out_specs,
        out_shape=out_shape,
    )(*args)
    return res if proj else res[0]


_MESH = plsc.VectorSubcoreMesh(core_axis_name="c", subcore_axis_name="s",
                               num_cores=NC, num_subcores=NS)


ZCH = 40
NZC = ROWS_PT // ZCH
NDEN = NPAD // 8
DEN_PT = NDEN // NS
NZC2 = DEN_PT // ZCH


@functools.partial(
    pl.kernel,
    mesh=_MESH,
    out_type=(jax.ShapeDtypeStruct((NC, NPAD, D), _f32),
              jax.ShapeDtypeStruct((NC, NDEN, D), _f32)),
    scratch_types=[
        pltpu.VMEM((EB,), jnp.int32),
        pltpu.VMEM((EB,), jnp.int32),
        pltpu.VMEM((EB,), jnp.int32),
        pltpu.VMEM((EB,), jnp.int32),
        pltpu.VMEM((EB, D), _f32),
        pltpu.VMEM((EB, D), _f32),
        pltpu.VMEM((EB, D), _f32),
        pltpu.VMEM((EB, D), _f32),
        pltpu.VMEM((NZC, ZCH), jnp.int32),
        pltpu.VMEM((NZC2, ZCH), jnp.int32),
        pltpu.VMEM_SHARED((NPAD, D), _f32),
        pltpu.VMEM_SHARED((NDEN, D), _f32),
        pltpu.SemaphoreType.DMA,
        pltpu.SemaphoreType.DMA,
        pltpu.SemaphoreType.DMA,
    ],
)
def _sc_edge_pass(q_hbm, k_hbm, v_hbm, src_hbm, dst_hbm, dst8_hbm, dst7_hbm,
                  ind_hbm, rows_hbm, rows2_hbm, zrow_hbm, tu_out,
                  den_out, sidx, didx, didx8, didx7, qbuf, kbuf,
                  vbuf, denrow, ridx, ridx2, acc_sh, den_sh,
                  sem_q, sem_k, sem_v):
    cid = lax.axis_index("c")
    sid = lax.axis_index("s")
    wid = cid * NS + sid
    r0 = sid * ROWS_PT
    r2 = sid * DEN_PT

    pltpu.sync_copy(rows_hbm.at[sid], ridx)
    pltpu.sync_copy(rows2_hbm.at[sid], ridx2)
    pltpu.sync_copy(zrow_hbm, denrow)
    for z in range(NZC):
        pltpu.sync_copy(denrow, acc_sh.at[ridx.at[z]])
    for z in range(NZC2):
        pltpu.sync_copy(denrow, den_sh.at[ridx2.at[z]])
    plsc.subcore_barrier()

    base = wid * EPW

    def block(bi, carry):
        eb = base + bi * EB
        pltpu.sync_copy(src_hbm.at[pl.ds(eb, EB)], sidx)
        pltpu.sync_copy(dst_hbm.at[pl.ds(eb, EB)], didx)
        pltpu.sync_copy(dst8_hbm.at[pl.ds(eb, EB)], didx8)
        pltpu.sync_copy(dst7_hbm.at[pl.ds(eb, EB)], didx7)
        cq = pltpu.async_copy(q_hbm.at[didx], qbuf, sem_q)
        ck = pltpu.async_copy(k_hbm.at[sidx], kbuf, sem_k)
        cv = pltpu.async_copy(v_hbm.at[sidx], vbuf, sem_v)
        ci = pltpu.async_copy(ind_hbm.at[didx7], denrow, sem_q)
        cq.wait()
        ck.wait()
        cv.wait()
        ci.wait()

        def edge(i, c):
            s = qbuf[i, pl.ds(0, 16)] * kbuf[i, pl.ds(0, 16)]
            for j in range(1, D // 16):
                s = s + qbuf[i, pl.ds(j * 16, 16)] * kbuf[i, pl.ds(j * 16, 16)]
            w = jnp.exp(s + lax.rev(s, (0,)))
            for j in range(D // 16):
                vbuf[i, pl.ds(j * 16, 16)] = w * vbuf[i, pl.ds(j * 16, 16)]
                denrow[i, pl.ds(j * 16, 16)] = w * denrow[i, pl.ds(j * 16, 16)]
            return c

        lax.fori_loop(0, EB, edge, 0)
        pltpu.sync_copy(vbuf, acc_sh.at[didx], add=True)
        pltpu.sync_copy(denrow, den_sh.at[didx8], add=True)
        return carry

    lax.fori_loop(0, NBLK, block, 0)
    plsc.subcore_barrier()

    for z in range(NZC):
        pltpu.async_copy(acc_sh.at[ridx.at[z]], denrow, sem_q).wait()
        pltpu.sync_copy(denrow, tu_out.at[cid, pl.ds(r0 + z * ZCH, ZCH)])
    for z in range(NZC2):
        pltpu.async_copy(den_sh.at[ridx2.at[z]], denrow, sem_q).wait()
        pltpu.sync_copy(denrow, den_out.at[cid, pl.ds(r2 + z * ZCH, ZCH)])


@functools.partial(
    pl.kernel,
    mesh=_MESH,
    out_type=jax.ShapeDtypeStruct((E2,), _f32),
    compiler_params=pltpu.CompilerParams(needs_layout_passes=False),
    scratch_types=[
        pltpu.VMEM((EB2,), jnp.int32),
        pltpu.VMEM((EB2,), jnp.int32),
        pltpu.VMEM((EB2, D), _f32),
        pltpu.VMEM((EB2, D), _f32),
        pltpu.VMEM((EB2,), _f32),
        pltpu.SemaphoreType.DMA,
        pltpu.SemaphoreType.DMA,
    ],
)
def _sc_decoder(h_hbm, src_hbm, dst_hbm, out_hbm, aidx, bidx, abuf, bbuf,
                sbuf, sem_a, sem_b):
    cid = lax.axis_index("c")
    sid = lax.axis_index("s")
    wid = cid * NS + sid
    base = wid * EPW2

    def block(bi, carry):
        eb = base + bi * EB2
        pltpu.sync_copy(src_hbm.at[pl.ds(eb, EB2)], aidx)
        pltpu.sync_copy(dst_hbm.at[pl.ds(eb, EB2)], bidx)
        ca = pltpu.async_copy(h_hbm.at[aidx], abuf, sem_a)
        cb = pltpu.async_copy(h_hbm.at[bidx], bbuf, sem_b)
        ca.wait()
        cb.wait()

        lane = lax.iota(jnp.int32, 16)
        last = lane == 15

        def edge(i, c):
            s = abuf[i, pl.ds(0, 16)] * bbuf[i, pl.ds(0, 16)]
            for j in range(1, D // 16):
                s = s + abuf[i, pl.ds(j * 16, 16)] * bbuf[i, pl.ds(j * 16, 16)]
            tot = plsc.cumsum(s)
            plsc.store_scatter(sbuf, [jnp.broadcast_to(i, (16,))], tot,
                               mask=last)
            return c

        lax.fori_loop(0, EB2, edge, 0)

        def sig(t, c):
            v = sbuf[pl.ds(t * 16, 16)]
            sbuf[pl.ds(t * 16, 16)] = 1.0 / (1.0 + jnp.exp(-v))
            return c

        lax.fori_loop(0, EB2 // 16, sig, 0)
        pltpu.sync_copy(sbuf, out_hbm.at[pl.ds(eb, EB2)])
        return carry

    lax.fori_loop(0, NBLK2, block, 0)


def kernel(x, edge_index, neg_edge_index, adapt_W, adapt_b, kW, kb, qW, qb,
           vW, vb, aW, ab, rel_pri, rel_att, rel_msg, skip, ln_g, ln_b):
    p = jnp.asarray(_P)
    scale = rel_pri / math.sqrt(DK)

    wq = [(qW[l].reshape(D, H, DK) * scale[l][None, :, None]).reshape(D, D)[:, p]
          for l in range(L)]
    bq = [((qb[l].reshape(H, DK) * scale[l][:, None]).reshape(D)[p]).reshape(1, D)
          for l in range(L)]
    wk = [jnp.einsum('ihd,hde->ihe', kW[l].reshape(D, H, DK),
                     rel_att[l]).reshape(D, D)[:, p] for l in range(L)]
    bk = [(jnp.einsum('hd,hde->he', kb[l].reshape(H, DK),
                      rel_att[l]).reshape(D)[p]).reshape(1, D) for l in range(L)]
    wv = [jnp.einsum('ihd,hde->ihe', vW[l].reshape(D, H, DK),
                     rel_msg[l]).reshape(D, D)[:, p] for l in range(L)]
    bv = [(jnp.einsum('hd,hde->he', vb[l].reshape(H, DK),
                      rel_msg[l]).reshape(D)[p]).reshape(1, D) for l in range(L)]
    wa = [aW[l][p, :] for l in range(L)]
    abr = [ab[l].reshape(1, D) for l in range(L)]
    g = [ln_g[l].reshape(1, D) for l in range(L)]
    bb = [ln_b[l].reshape(1, D) for l in range(L)]
    alpha = [jax.nn.sigmoid(skip[l]).reshape(1, 1) for l in range(L)]

    src = edge_index[0]
    dst = edge_index[1]
    csrc = jnp.concatenate([edge_index[0], neg_edge_index[0]])
    cdst = jnp.concatenate([edge_index[1], neg_edge_index[1]])

    dst8 = dst >> 3
    dst7 = dst & 7
    rows = jnp.arange(NPAD, dtype=jnp.int32).reshape(NS, NZC, ZCH)
    rows2 = jnp.arange(NDEN, dtype=jnp.int32).reshape(NS, NZC2, ZCH)
    zrow = jnp.zeros((EB, D), _f32)
    ind = jnp.repeat(jnp.eye(8, dtype=_f32), 16, axis=1)

    h, q, k, v = _tc_adapt(x, adapt_W, adapt_b.reshape(1, D),
                           wq[0], bq[0], wk[0], bk[0], wv[0], bv[0])
    tu, den = _sc_edge_pass(q, k, v, src, dst, dst8, dst7, ind, rows, rows2, zrow)
    den = den.reshape(NC, NPAD, 16)
    h, q, k, v = _tc_combine(tu, den, h, wa[0], abr[0], g[0], bb[0], alpha[0],
                             proj_w=(wq[1], bq[1], wk[1], bk[1], wv[1], bv[1]))
    tu, den = _sc_edge_pass(q, k, v, src, dst, dst8, dst7, ind, rows, rows2, zrow)
    den = den.reshape(NC, NPAD, 16)
    h = _tc_combine(tu, den, h, wa[1], abr[1], g[1], bb[1], alpha[1])
    return _sc_decoder(h, csrc, cdst)


# speedup vs baseline: 19.0219x; 1.0096x over previous
import functools
import math

import jax
import jax.numpy as jnp
import numpy as np
from jax import lax
from jax.experimental import pallas as pl
from jax.experimental.pallas import tpu as pltpu
from jax.experimental.pallas import tpu_sc as plsc

N = 10000
E = 320000
D = 128
H = 8
DK = 16
L = 2

NC = 2
NS = 16
NW = NC * NS

NPAD = 10240
EPW = E // NW
EB = 40
NBLK = EPW // EB
ROWS_PT = NPAD // NS
ZR = 128

E2 = 2 * E
EPW2 = E2 // NW
EB2 = 80
NBLK2 = EPW2 // EB2

RB = 1000
NRB = N // RB

_f32 = jnp.float32


def _perm():
    p = np.zeros(D, np.int32)
    for c in range(D):
        j, l = divmod(c, 16)
        head = l if l < 8 else 15 - l
        dk = j if l < 8 else j + 8
        p[c] = head * 16 + dk
    return p


_P = _perm()


def _tc_adapt_body(x_ref, wa_ref, ba_ref, wq_ref, bq_ref, wk_ref, bk_ref,
                   wv_ref, bv_ref, h_ref, q_ref, k_ref, v_ref):
    xb = x_ref[...]
    pre = xb @ wa_ref[...] + ba_ref[...]
    hb = 0.5 * pre * (1.0 + lax.erf(pre * (1.0 / math.sqrt(2.0))))
    h_ref[...] = hb
    q_ref[...] = hb @ wq_ref[...] + bq_ref[...]
    k_ref[...] = hb @ wk_ref[...] + bk_ref[...]
    v_ref[...] = hb @ wv_ref[...] + bv_ref[...]


def _tc_adapt(x, wa, ba, wq, bq, wk, bk, wv, bv):
    row = pl.BlockSpec((RB, D), lambda i: (i, 0))
    wsp = pl.BlockSpec((D, D), lambda i: (0, 0))
    bsp = pl.BlockSpec((1, D), lambda i: (0, 0))
    out = jax.ShapeDtypeStruct((N, D), _f32)
    return pl.pallas_call(
        _tc_adapt_body,
        grid=(NRB,),
        in_specs=[row, wsp, bsp, wsp, bsp, wsp, bsp, wsp, bsp],
        out_specs=[row, row, row, row],
        out_shape=[out, out, out, out],
    )(x, wa, ba, wq, bq, wk, bk, wv, bv)


def _tc_combine_body(proj, tu_ref, den_ref, h_ref, wa_ref, ab_ref, g_ref,
                     b_ref, sc_ref, *rest):
    tu = tu_ref[0] + tu_ref[1]
    den = den_ref[0] + den_ref[1]
    den = jnp.maximum(jnp.concatenate([den] * (D // 16), axis=-1), 1e-9)
    t = tu / den
    alpha = sc_ref[0, 0]
    out = (t @ wa_ref[...] + ab_ref[...]) * alpha + h_ref[...] * (1.0 - alpha)
    mu = jnp.mean(out, axis=-1, keepdims=True)
    var = jnp.mean((out - mu) * (out - mu), axis=-1, keepdims=True)
    hb = (out - mu) * jax.lax.rsqrt(var + 1e-5) * g_ref[...] + b_ref[...]
    if proj:
        (wq_ref, bq_ref, wk_ref, bk_ref, wv_ref, bv_ref,
         h_out, q_out, k_out, v_out) = rest
        h_out[...] = hb
        q_out[...] = hb @ wq_ref[...] + bq_ref[...]
        k_out[...] = hb @ wk_ref[...] + bk_ref[...]
        v_out[...] = hb @ wv_ref[...] + bv_ref[...]
    else:
        (h_out,) = rest
        h_out[...] = hb


def _tc_combine(tu_parts, den_parts, h, wa, ab, g, b, alpha, proj_w=None):
    row = pl.BlockSpec((RB, D), lambda i: (i, 0))
    wsp = pl.BlockSpec((D, D), lambda i: (0, 0))
    bsp = pl.BlockSpec((1, D), lambda i: (0, 0))
    ssp = pl.BlockSpec((1, 1), lambda i: (0, 0))
    tusp = pl.BlockSpec((2, RB, D), lambda i: (0, i, 0))
    densp = pl.BlockSpec((2, RB, 16), lambda i: (0, i, 0))
    out = jax.ShapeDtypeStruct((N, D), _f32)
    proj = proj_w is not None
    in_specs = [tusp, densp, row, wsp, bsp, bsp, bsp, ssp]
    args = [tu_parts, den_parts, h, wa, ab, g, b, alpha]
    if proj:
        in_specs += [wsp, bsp, wsp, bsp, wsp, bsp]
        args += list(proj_w)
        out_specs, out_shape = [row] * 4, [out] * 4
    else:
        out_specs, out_shape = [row], [out]
    res = pl.pallas_call(
        functools.partial(_tc_combine_body, proj),
        grid=(NRB,),
        in_specs=in_specs,
        out_specs=out_specs,
        out_shape=out_shape,
    )(*args)
    return res if proj else res[0]


_MESH = plsc.VectorSubcoreMesh(core_axis_name="c", subcore_axis_name="s",
                               num_cores=NC, num_subcores=NS)


ZCH = 40
NZC = ROWS_PT // ZCH
NDEN = NPAD // 8
DEN_PT = NDEN // NS
NZC2 = DEN_PT // ZCH


@functools.partial(
    pl.kernel,
    mesh=_MESH,
    out_type=(jax.ShapeDtypeStruct((NC, NPAD, D), _f32),
              jax.ShapeDtypeStruct((NC, NDEN, D), _f32)),
    scratch_types=[
        pltpu.VMEM((EB,), jnp.int32),
        pltpu.VMEM((EB,), jnp.int32),
        pltpu.VMEM((EB,), jnp.int32),
        pltpu.VMEM((EB,), jnp.int32),
        pltpu.VMEM((EB,), jnp.int32),
        pltpu.VMEM((EB,), jnp.int32),
        pltpu.VMEM((EB,), jnp.int32),
        pltpu.VMEM((EB,), jnp.int32),
        pltpu.VMEM((EB, D), _f32),
        pltpu.VMEM((EB, D), _f32),
        pltpu.VMEM((EB, D), _f32),
        pltpu.VMEM((EB, D), _f32),
        pltpu.VMEM((EB, D), _f32),
        pltpu.VMEM((EB, D), _f32),
        pltpu.VMEM((NZC, ZCH), jnp.int32),
        pltpu.VMEM((NZC2, ZCH), jnp.int32),
        pltpu.VMEM_SHARED((NPAD, D), _f32),
        pltpu.VMEM_SHARED((NDEN, D), _f32),
        pltpu.SemaphoreType.DMA,
        pltpu.SemaphoreType.DMA,
        pltpu.SemaphoreType.DMA,
        pltpu.SemaphoreType.DMA,
        pltpu.SemaphoreType.DMA,
        pltpu.SemaphoreType.DMA,
    ],
)
def _sc_edge_pass(q_hbm, k_hbm, v_hbm, src_hbm, dst_hbm, dst8_hbm, dst7_hbm,
                  ind_hbm, rows_hbm, rows2_hbm, zrow_hbm, tu_out, den_out,
                  sidx0, didx0, didx80, didx70, sidx1, didx1, didx81, didx71,
                  qbuf0, kbuf0, qbuf1, kbuf1, vbuf, denrow,
                  ridx, ridx2, acc_sh, den_sh,
                  sq0, sk0, sq1, sk1, sv, si):
    cid = lax.axis_index("c")
    sid = lax.axis_index("s")
    wid = cid * NS + sid
    r0 = sid * ROWS_PT
    r2 = sid * DEN_PT
    base = wid * EPW

    idx = ((sidx0, didx0, didx80, didx70), (sidx1, didx1, didx81, didx71))
    bufs = ((qbuf0, kbuf0), (qbuf1, kbuf1))
    sems = ((sq0, sk0), (sq1, sk1))

    pltpu.sync_copy(rows_hbm.at[sid], ridx)
    pltpu.sync_copy(rows2_hbm.at[sid], ridx2)
    pltpu.sync_copy(zrow_hbm, denrow)
    for z in range(NZC):
        pltpu.sync_copy(denrow, acc_sh.at[ridx.at[z]])
    for z in range(NZC2):
        pltpu.sync_copy(denrow, den_sh.at[ridx2.at[z]])
    plsc.subcore_barrier()

    def fetch_idx(b, p):
        eb = base + b * EB
        si_, di_, d8_, d7_ = idx[p]
        pltpu.sync_copy(src_hbm.at[pl.ds(eb, EB)], si_)
        pltpu.sync_copy(dst_hbm.at[pl.ds(eb, EB)], di_)
        pltpu.sync_copy(dst8_hbm.at[pl.ds(eb, EB)], d8_)
        pltpu.sync_copy(dst7_hbm.at[pl.ds(eb, EB)], d7_)

    def fire(p):
        si_, di_, d8_, d7_ = idx[p]
        qb, kb = bufs[p]
        q_s, k_s = sems[p]
        pltpu.async_copy(q_hbm.at[di_], qb, q_s)
        pltpu.async_copy(k_hbm.at[si_], kb, k_s)

    def fire_v(p):
        pltpu.async_copy(v_hbm.at[idx[p][0]], vbuf, sv)

    def fire_ind(p):
        pltpu.async_copy(ind_hbm.at[idx[p][3]], denrow, si)

    def drain(p):
        qb, kb = bufs[p]
        q_s, k_s = sems[p]
        pltpu.make_async_copy(q_hbm.at[pl.ds(0, EB)], qb, q_s).wait()
        pltpu.make_async_copy(k_hbm.at[pl.ds(0, EB)], kb, k_s).wait()
        pltpu.make_async_copy(v_hbm.at[pl.ds(0, EB)], vbuf, sv).wait()
        pltpu.make_async_copy(ind_hbm.at[pl.ds(0, EB)], denrow, si).wait()

    def compute_and_scatter(p, issue_next):
        qb, kb = bufs[p]
        drain(p)

        def edge(i, c):
            s = qb[i, pl.ds(0, 16)] * kb[i, pl.ds(0, 16)]
            for j in range(1, D // 16):
                s = s + qb[i, pl.ds(j * 16, 16)] * kb[i, pl.ds(j * 16, 16)]
            w = jnp.exp(s + lax.rev(s, (0,)))
            for j in range(D // 16):
                vbuf[i, pl.ds(j * 16, 16)] = w * vbuf[i, pl.ds(j * 16, 16)]
                denrow[i, pl.ds(j * 16, 16)] = w * denrow[i, pl.ds(j * 16, 16)]
            return c

        lax.fori_loop(0, EB, edge, 0)
        pltpu.sync_copy(denrow, den_sh.at[idx[p][2]], add=True)
        if issue_next:
            fire_ind(1 - p)
        pltpu.sync_copy(vbuf, acc_sh.at[idx[p][1]], add=True)
        if issue_next:
            fire_v(1 - p)

    fetch_idx(0, 0)
    fire(0)
    fire_v(0)
    fire_ind(0)

    def pair(g, carry):
        fetch_idx(2 * g + 1, 1)
        fire(1)
        compute_and_scatter(0, True)
        fetch_idx(2 * g + 2, 0)
        fire(0)
        compute_and_scatter(1, True)
        return carry

    lax.fori_loop(0, NBLK // 2 - 1, pair, 0)
    fetch_idx(NBLK - 1, 1)
    fire(1)
    compute_and_scatter(0, True)
    compute_and_scatter(1, False)
    plsc.subcore_barrier()

    for z in range(NZC):
        pltpu.async_copy(acc_sh.at[ridx.at[z]], denrow, sq0).wait()
        pltpu.sync_copy(denrow, tu_out.at[cid, pl.ds(r0 + z * ZCH, ZCH)])
    for z in range(NZC2):
        pltpu.async_copy(den_sh.at[ridx2.at[z]], denrow, sq0).wait()
        pltpu.sync_copy(denrow, den_out.at[cid, pl.ds(r2 + z * ZCH, ZCH)])


@functools.partial(
    pl.kernel,
    mesh=_MESH,
    out_type=jax.ShapeDtypeStruct((E2,), _f32),
    compiler_params=pltpu.CompilerParams(needs_layout_passes=False),
    scratch_types=[
        pltpu.VMEM((EB2,), jnp.int32),
        pltpu.VMEM((EB2,), jnp.int32),
        pltpu.VMEM((EB2, D), _f32),
        pltpu.VMEM((EB2, D), _f32),
        pltpu.VMEM((EB2,), _f32),
        pltpu.SemaphoreType.DMA,
        pltpu.SemaphoreType.DMA,
    ],
)
def _sc_decoder(h_hbm, src_hbm, dst_hbm, out_hbm, aidx, bidx, abuf, bbuf,
                sbuf, sem_a, sem_b):
    cid = lax.axis_index("c")
    sid = lax.axis_index("s")
    wid = cid * NS + sid
    base = wid * EPW2

    def block(bi, carry):
        eb = base + bi * EB2
        pltpu.sync_copy(src_hbm.at[pl.ds(eb, EB2)], aidx)
        pltpu.sync_copy(dst_hbm.at[pl.ds(eb, EB2)], bidx)
        ca = pltpu.async_copy(h_hbm.at[aidx], abuf, sem_a)
        cb = pltpu.async_copy(h_hbm.at[bidx], bbuf, sem_b)
        ca.wait()
        cb.wait()

        lane = lax.iota(jnp.int32, 16)
        last = lane == 15

        def edge(i, c):
            s = abuf[i, pl.ds(0, 16)] * bbuf[i, pl.ds(0, 16)]
            for j in range(1, D // 16):
                s = s + abuf[i, pl.ds(j * 16, 16)] * bbuf[i, pl.ds(j * 16, 16)]
            tot = plsc.cumsum(s)
            plsc.store_scatter(sbuf, [jnp.broadcast_to(i, (16,))], tot,
                               mask=last)
            return c

        lax.fori_loop(0, EB2, edge, 0)

        def sig(t, c):
            v = sbuf[pl.ds(t * 16, 16)]
            sbuf[pl.ds(t * 16, 16)] = 1.0 / (1.0 + jnp.exp(-v))
            return c

        lax.fori_loop(0, EB2 // 16, sig, 0)
        pltpu.sync_copy(sbuf, out_hbm.at[pl.ds(eb, EB2)])
        return carry

    lax.fori_loop(0, NBLK2, block, 0)


def kernel(x, edge_index, neg_edge_index, adapt_W, adapt_b, kW, kb, qW, qb,
           vW, vb, aW, ab, rel_pri, rel_att, rel_msg, skip, ln_g, ln_b):
    p = jnp.asarray(_P)
    scale = rel_pri / math.sqrt(DK)

    wq = [(qW[l].reshape(D, H, DK) * scale[l][None, :, None]).reshape(D, D)[:, p]
          for l in range(L)]
    bq = [((qb[l].reshape(H, DK) * scale[l][:, None]).reshape(D)[p]).reshape(1, D)
          for l in range(L)]
    wk = [jnp.einsum('ihd,hde->ihe', kW[l].reshape(D, H, DK),
                     rel_att[l]).reshape(D, D)[:, p] for l in range(L)]
    bk = [(jnp.einsum('hd,hde->he', kb[l].reshape(H, DK),
                      rel_att[l]).reshape(D)[p]).reshape(1, D) for l in range(L)]
    wv = [jnp.einsum('ihd,hde->ihe', vW[l].reshape(D, H, DK),
                     rel_msg[l]).reshape(D, D)[:, p] for l in range(L)]
    bv = [(jnp.einsum('hd,hde->he', vb[l].reshape(H, DK),
                      rel_msg[l]).reshape(D)[p]).reshape(1, D) for l in range(L)]
    wa = [aW[l][p, :] for l in range(L)]
    abr = [ab[l].reshape(1, D) for l in range(L)]
    g = [ln_g[l].reshape(1, D) for l in range(L)]
    bb = [ln_b[l].reshape(1, D) for l in range(L)]
    alpha = [jax.nn.sigmoid(skip[l]).reshape(1, 1) for l in range(L)]

    src = edge_index[0]
    dst = edge_index[1]
    csrc = jnp.concatenate([edge_index[0], neg_edge_index[0]])
    cdst = jnp.concatenate([edge_index[1], neg_edge_index[1]])

    dst8 = dst >> 3
    dst7 = dst & 7
    rows = jnp.arange(NPAD, dtype=jnp.int32).reshape(NS, NZC, ZCH)
    rows2 = jnp.arange(NDEN, dtype=jnp.int32).reshape(NS, NZC2, ZCH)
    zrow = jnp.zeros((EB, D), _f32)
    ind = jnp.repeat(jnp.eye(8, dtype=_f32), 16, axis=1)

    h, q, k, v = _tc_adapt(x, adapt_W, adapt_b.reshape(1, D),
                           wq[0], bq[0], wk[0], bk[0], wv[0], bv[0])
    tu, den = _sc_edge_pass(q, k, v, src, dst, dst8, dst7, ind, rows, rows2, zrow)
    den = den.reshape(NC, NPAD, 16)
    h, q, k, v = _tc_combine(tu, den, h, wa[0], abr[0], g[0], bb[0], alpha[0],
                             proj_w=(wq[1], bq[1], wk[1], bk[1], wv[1], bv[1]))
    tu, den = _sc_edge_pass(q, k, v, src, dst, dst8, dst7, ind, rows, rows2, zrow)
    den = den.reshape(NC, NPAD, 16)
    h = _tc_combine(tu, den, h, wa[1], abr[1], g[1], bb[1], alpha[1])
    return _sc_decoder(h, csrc, cdst)
